# Initial kernel scaffold; baseline (speedup 1.0000x reference)
import jax
import jax.numpy as jnp
from jax.experimental import pallas as pl


def kernel(x, edge_index, batch, W1, b1, g1, bt1, W2, b2, g2, bt2, W3, b3, g3, bt3, Wl, bl):
    raise NotImplementedError("write your pallas kernel here")


# speedup vs baseline: 23.1329x; 23.1329x over previous
import functools

import jax
import jax.numpy as jnp
import numpy as np
from jax import lax
from jax.experimental import pallas as pl
from jax.experimental.pallas import tpu as pltpu
from jax.experimental.pallas import tpu_sc as plsc

N = 10000
NP = 10240
D = 128
E = 320000
G = 64
NCLS = 4
NTILES = 32
WSZ = 128
WPC = 160
WPT = 80
EPT = WPC * WSZ
EPAD = 16 * EPT
DH = D // 2
RPT = NP // 16

def _deg_body(dst_hbm, out_hbm, dst_v, ones_v, zer_v, acc_s):
    cid = lax.axis_index("c")
    sid = lax.axis_index("s")
    pltpu.sync_copy(dst_hbm.at[sid, pl.ds(cid * WPT, WPT)], dst_v)
    zeros16 = jnp.zeros((16,), jnp.float32)
    for i in range(RPT // 16):
        zer_v[pl.ds(i * 16, 16)] = zeros16
    ones16 = jnp.ones((16,), jnp.float32)
    for i in range(WSZ // 16):
        ones_v[pl.ds(i * 16, 16)] = ones16
    pltpu.sync_copy(zer_v, acc_s.at[pl.ds(sid * RPT, RPT)])
    plsc.subcore_barrier()
    for w in range(WPT):
        pltpu.sync_copy(ones_v, acc_s.at[dst_v.at[np.int32(w)]], add=True)
    plsc.subcore_barrier()
    pltpu.sync_copy(acc_s.at[pl.ds(sid * RPT, RPT)],
                    out_hbm.at[cid, pl.ds(sid * RPT, RPT)])


@functools.cache
def _deg_call():
    mesh = plsc.VectorSubcoreMesh(core_axis_name="c", subcore_axis_name="s",
                                  num_cores=2, num_subcores=16)
    return pl.kernel(
        _deg_body,
        out_type=jax.ShapeDtypeStruct((2, NP), jnp.float32),
        mesh=mesh,
        scratch_types=[
            pltpu.VMEM((WPT, WSZ), jnp.int32),
            pltpu.VMEM((WSZ,), jnp.float32),
            pltpu.VMEM((RPT,), jnp.float32),
            pltpu.VMEM_SHARED((NP,), jnp.float32),
        ],
    )


def _scat_body(hs_hbm, srcw_hbm, dstw_hbm, out_hbm,
               src_v, dst_v, buf_v, sem0, sem1, acc_s):
    cid = lax.axis_index("c")
    sid = lax.axis_index("s")
    pltpu.sync_copy(srcw_hbm.at[sid], src_v)
    pltpu.sync_copy(dstw_hbm.at[sid], dst_v)
    pltpu.sync_copy(hs_hbm.at[cid, pl.ds(sid * RPT, RPT)],
                    acc_s.at[pl.ds(sid * RPT, RPT)])
    plsc.subcore_barrier()
    sems = (sem0, sem1)
    cps = [None, None]
    cps[0] = pltpu.async_copy(hs_hbm.at[cid].at[src_v.at[np.int32(0)]],
                              buf_v.at[np.int32(0)], sems[0])
    for w in range(WPC):
        if w + 1 < WPC:
            cps[(w + 1) % 2] = pltpu.async_copy(
                hs_hbm.at[cid].at[src_v.at[np.int32(w + 1)]],
                buf_v.at[np.int32((w + 1) % 2)], sems[(w + 1) % 2])
        cps[w % 2].wait()
        pltpu.sync_copy(buf_v.at[np.int32(w % 2)],
                        acc_s.at[dst_v.at[np.int32(w)]], add=True)
    plsc.subcore_barrier()
    pltpu.sync_copy(acc_s.at[pl.ds(sid * RPT, RPT)],
                    out_hbm.at[cid, pl.ds(sid * RPT, RPT)])


@functools.cache
def _scat_call():
    mesh = plsc.VectorSubcoreMesh(core_axis_name="c", subcore_axis_name="s",
                                  num_cores=2, num_subcores=16)
    return pl.kernel(
        _scat_body,
        out_type=jax.ShapeDtypeStruct((2, NP, DH), jnp.float32),
        mesh=mesh,
        compiler_params=pltpu.CompilerParams(use_tc_tiling_on_sc=False),
        scratch_types=[
            pltpu.VMEM((WPC, WSZ), jnp.int32),
            pltpu.VMEM((WPC, WSZ), jnp.int32),
            pltpu.VMEM((2, WSZ, DH), jnp.float32),
            pltpu.SemaphoreType.DMA,
            pltpu.SemaphoreType.DMA,
            pltpu.VMEM_SHARED((NP, DH), jnp.float32),
        ],
    )


def _k1_body(x_ref, w_ref, degp_ref, h_ref, dis_ref):
    d = 1.0 + degp_ref[:, 0:1] + degp_ref[:, 1:2]
    dis = lax.rsqrt(d)
    h = jnp.dot(x_ref[...], w_ref[...],
                preferred_element_type=jnp.float32) * dis
    h_ref[0] = h[:, :DH]
    h_ref[1] = h[:, DH:]
    dis_ref[...] = dis


_k1_call = pl.pallas_call(
    _k1_body,
    out_shape=[jax.ShapeDtypeStruct((2, NP, DH), jnp.float32),
               jax.ShapeDtypeStruct((NP, 1), jnp.float32)],
)


def _bn_relu(p_ref, dis_ref, b_ref, g_ref, bt_ref):
    agg = jnp.concatenate([p_ref[0], p_ref[1]], axis=1)
    a = dis_ref[...] * agg + b_ref[...]
    mask = (lax.broadcasted_iota(jnp.int32, (NP, 1), 0) < N)
    am = jnp.where(mask, a, 0.0)
    s1 = jnp.sum(am, axis=0, keepdims=True)
    s2 = jnp.sum(am * am, axis=0, keepdims=True)
    mu = s1 * (1.0 / N)
    var = s2 * (1.0 / N) - mu * mu
    return jnp.maximum((a - mu) * lax.rsqrt(var + 1e-5) * g_ref[...]
                       + bt_ref[...], 0.0)


def _kmid_body(p_ref, dis_ref, b_ref, g_ref, bt_ref, w_ref, o_ref):
    r = _bn_relu(p_ref, dis_ref, b_ref, g_ref, bt_ref)
    h = jnp.dot(r, w_ref[...],
                preferred_element_type=jnp.float32) * dis_ref[...]
    o_ref[0] = h[:, :DH]
    o_ref[1] = h[:, DH:]


_kmid_call = pl.pallas_call(
    _kmid_body,
    out_shape=jax.ShapeDtypeStruct((2, NP, DH), jnp.float32),
)


def _kfin_body(p_ref, dis_ref, b_ref, g_ref, bt_ref, batch_ref,
               wl_ref, bl_ref, o_ref):
    r = _bn_relu(p_ref, dis_ref, b_ref, g_ref, bt_ref)
    r10 = r[:N, :]
    oh = (batch_ref[...] ==
          lax.broadcasted_iota(jnp.int32, (G, N), 0)).astype(jnp.float32)
    sums = jnp.dot(oh, r10, preferred_element_type=jnp.float32)
    cnts = jnp.sum(oh, axis=1, keepdims=True)
    pooled = sums / jnp.maximum(cnts, 1.0)
    o_ref[...] = jnp.dot(pooled, wl_ref[...],
                         preferred_element_type=jnp.float32) + bl_ref[...]


_kfin_call = pl.pallas_call(
    _kfin_body,
    out_shape=jax.ShapeDtypeStruct((G, NCLS), jnp.float32),
)


def kernel(x, edge_index, batch, W1, b1, g1, bt1, W2, b2, g2, bt2,
           W3, b3, g3, bt3, Wl, bl):
    src = edge_index[0].astype(jnp.int32)
    dst = edge_index[1].astype(jnp.int32)
    npad = EPAD - E
    ar = jnp.arange(npad, dtype=jnp.int32)
    srcw = jnp.concatenate([src, (ar * 911) % N]).reshape(16, WPC, WSZ)
    dstw = jnp.concatenate([dst, N + ar % (NP - N)]).reshape(16, WPC, WSZ)
    xp = jnp.pad(x, ((0, NP - N), (0, 0)))
    batch2d = batch.astype(jnp.int32).reshape(1, N)
    b1r, g1r, bt1r = b1.reshape(1, D), g1.reshape(1, D), bt1.reshape(1, D)
    b2r, g2r, bt2r = b2.reshape(1, D), g2.reshape(1, D), bt2.reshape(1, D)
    b3r, g3r, bt3r = b3.reshape(1, D), g3.reshape(1, D), bt3.reshape(1, D)

    degp = _deg_call()(dstw)
    h1, dis = _k1_call(xp, W1, degp.T)
    p1 = _scat_call()(h1, srcw, dstw)
    h2 = _kmid_call(p1, dis, b1r, g1r, bt1r, W2)
    p2 = _scat_call()(h2, srcw, dstw)
    h3 = _kmid_call(p2, dis, b2r, g2r, bt2r, W3)
    p3 = _scat_call()(h3, srcw, dstw)
    return _kfin_call(p3, dis, b3r, g3r, bt3r, batch2d, Wl,
                      bl.reshape(1, NCLS))


# speedup vs baseline: 28.1734x; 1.2179x over previous
import functools

import jax
import jax.numpy as jnp
import numpy as np
from jax import lax
from jax.experimental import pallas as pl
from jax.experimental.pallas import tpu as pltpu
from jax.experimental.pallas import tpu_sc as plsc

N = 10000
NP = 10240
D = 128
E = 320000
G = 64
NCLS = 4
NTILES = 32
WSZ = 128
WPC = 160
WPT = 80
EPT = WPC * WSZ
EPAD = 16 * EPT
DH = D // 2
RPT = NP // 16
NBUF = 6
LEAD = 3

def _deg_body(dst_hbm, out_hbm, dst_v, ones_v, zer_v, acc_s):
    cid = lax.axis_index("c")
    sid = lax.axis_index("s")
    pltpu.sync_copy(dst_hbm.at[sid, pl.ds(cid * WPT, WPT)], dst_v)
    zeros16 = jnp.zeros((16,), jnp.float32)
    for i in range(RPT // 16):
        zer_v[pl.ds(i * 16, 16)] = zeros16
    ones16 = jnp.ones((16,), jnp.float32)
    for i in range(WSZ // 16):
        ones_v[pl.ds(i * 16, 16)] = ones16
    pltpu.sync_copy(zer_v, acc_s.at[pl.ds(sid * RPT, RPT)])
    plsc.subcore_barrier()
    for w in range(WPT):
        pltpu.sync_copy(ones_v, acc_s.at[dst_v.at[np.int32(w)]], add=True)
    plsc.subcore_barrier()
    pltpu.sync_copy(acc_s.at[pl.ds(sid * RPT, RPT)],
                    out_hbm.at[cid, pl.ds(sid * RPT, RPT)])


@functools.cache
def _deg_call():
    mesh = plsc.VectorSubcoreMesh(core_axis_name="c", subcore_axis_name="s",
                                  num_cores=2, num_subcores=16)
    return pl.kernel(
        _deg_body,
        out_type=jax.ShapeDtypeStruct((2, NP), jnp.float32),
        mesh=mesh,
        scratch_types=[
            pltpu.VMEM((WPT, WSZ), jnp.int32),
            pltpu.VMEM((WSZ,), jnp.float32),
            pltpu.VMEM((RPT,), jnp.float32),
            pltpu.VMEM_SHARED((NP,), jnp.float32),
        ],
    )


def _scat_body(hs_hbm, srcw_hbm, dstw_hbm, out_hbm,
               src_v, dst_v, buf_v, gsem_a, ssem_a, acc_s):
    gsems = [gsem_a.at[np.int32(b)] for b in range(NBUF)]
    ssems = [ssem_a.at[np.int32(b)] for b in range(NBUF)]
    cid = lax.axis_index("c")
    sid = lax.axis_index("s")
    pltpu.sync_copy(srcw_hbm.at[sid], src_v)
    pltpu.sync_copy(dstw_hbm.at[sid], dst_v)
    pltpu.sync_copy(hs_hbm.at[cid, pl.ds(sid * RPT, RPT)],
                    acc_s.at[pl.ds(sid * RPT, RPT)])
    plsc.subcore_barrier()
    gcp = [None] * NBUF
    scp = [None] * NBUF

    def start_gather(w):
        b = w % NBUF
        gcp[b] = pltpu.async_copy(hs_hbm.at[cid].at[src_v.at[np.int32(w)]],
                                  buf_v.at[np.int32(b)], gsems[b])

    for w in range(LEAD):
        start_gather(w)
    for w in range(WPC):
        b = w % NBUF
        nxt = w + LEAD
        if nxt < WPC:
            nb = nxt % NBUF
            if scp[nb] is not None:
                scp[nb].wait()
                scp[nb] = None
            start_gather(nxt)
        gcp[b].wait()
        scp[b] = pltpu.async_copy(buf_v.at[np.int32(b)],
                                  acc_s.at[dst_v.at[np.int32(w)]],
                                  ssems[b], add=True)
    for b in range(NBUF):
        if scp[b] is not None:
            scp[b].wait()
    plsc.subcore_barrier()
    pltpu.sync_copy(acc_s.at[pl.ds(sid * RPT, RPT)],
                    out_hbm.at[cid, pl.ds(sid * RPT, RPT)])


@functools.cache
def _scat_call():
    mesh = plsc.VectorSubcoreMesh(core_axis_name="c", subcore_axis_name="s",
                                  num_cores=2, num_subcores=16)
    return pl.kernel(
        _scat_body,
        out_type=jax.ShapeDtypeStruct((2, NP, DH), jnp.float32),
        mesh=mesh,
        compiler_params=pltpu.CompilerParams(use_tc_tiling_on_sc=False),
        scratch_types=[
            pltpu.VMEM((WPC, WSZ), jnp.int32),
            pltpu.VMEM((WPC, WSZ), jnp.int32),
            pltpu.VMEM((NBUF, WSZ, DH), jnp.float32),
            pltpu.SemaphoreType.DMA((NBUF,)),
            pltpu.SemaphoreType.DMA((NBUF,)),
            pltpu.VMEM_SHARED((NP, DH), jnp.float32),
        ],
    )


def _k1_body(x_ref, w_ref, degp_ref, h_ref, dis_ref):
    d = 1.0 + degp_ref[:, 0:1] + degp_ref[:, 1:2]
    dis = lax.rsqrt(d)
    h = jnp.dot(x_ref[...], w_ref[...],
                preferred_element_type=jnp.float32) * dis
    h_ref[0] = h[:, :DH]
    h_ref[1] = h[:, DH:]
    dis_ref[...] = dis


_k1_call = pl.pallas_call(
    _k1_body,
    out_shape=[jax.ShapeDtypeStruct((2, NP, DH), jnp.float32),
               jax.ShapeDtypeStruct((NP, 1), jnp.float32)],
)


def _bn_relu(p_ref, dis_ref, b_ref, g_ref, bt_ref):
    agg = jnp.concatenate([p_ref[0], p_ref[1]], axis=1)
    a = dis_ref[...] * agg + b_ref[...]
    mask = (lax.broadcasted_iota(jnp.int32, (NP, 1), 0) < N)
    am = jnp.where(mask, a, 0.0)
    s1 = jnp.sum(am, axis=0, keepdims=True)
    s2 = jnp.sum(am * am, axis=0, keepdims=True)
    mu = s1 * (1.0 / N)
    var = s2 * (1.0 / N) - mu * mu
    return jnp.maximum((a - mu) * lax.rsqrt(var + 1e-5) * g_ref[...]
                       + bt_ref[...], 0.0)


def _kmid_body(p_ref, dis_ref, b_ref, g_ref, bt_ref, w_ref, o_ref):
    r = _bn_relu(p_ref, dis_ref, b_ref, g_ref, bt_ref)
    h = jnp.dot(r, w_ref[...],
                preferred_element_type=jnp.float32) * dis_ref[...]
    o_ref[0] = h[:, :DH]
    o_ref[1] = h[:, DH:]


_kmid_call = pl.pallas_call(
    _kmid_body,
    out_shape=jax.ShapeDtypeStruct((2, NP, DH), jnp.float32),
)


def _kfin_body(p_ref, dis_ref, b_ref, g_ref, bt_ref, batch_ref,
               wl_ref, bl_ref, o_ref):
    r = _bn_relu(p_ref, dis_ref, b_ref, g_ref, bt_ref)
    r10 = r[:N, :]
    oh = (batch_ref[...] ==
          lax.broadcasted_iota(jnp.int32, (G, N), 0)).astype(jnp.float32)
    sums = jnp.dot(oh, r10, preferred_element_type=jnp.float32)
    cnts = jnp.sum(oh, axis=1, keepdims=True)
    pooled = sums / jnp.maximum(cnts, 1.0)
    o_ref[...] = jnp.dot(pooled, wl_ref[...],
                         preferred_element_type=jnp.float32) + bl_ref[...]


_kfin_call = pl.pallas_call(
    _kfin_body,
    out_shape=jax.ShapeDtypeStruct((G, NCLS), jnp.float32),
)


def kernel(x, edge_index, batch, W1, b1, g1, bt1, W2, b2, g2, bt2,
           W3, b3, g3, bt3, Wl, bl):
    src = edge_index[0].astype(jnp.int32)
    dst = edge_index[1].astype(jnp.int32)
    npad = EPAD - E
    ar = jnp.arange(npad, dtype=jnp.int32)
    srcw = jnp.concatenate([src, (ar * 911) % N]).reshape(16, WPC, WSZ)
    dstw = jnp.concatenate([dst, N + ar % (NP - N)]).reshape(16, WPC, WSZ)
    xp = jnp.pad(x, ((0, NP - N), (0, 0)))
    batch2d = batch.astype(jnp.int32).reshape(1, N)
    b1r, g1r, bt1r = b1.reshape(1, D), g1.reshape(1, D), bt1.reshape(1, D)
    b2r, g2r, bt2r = b2.reshape(1, D), g2.reshape(1, D), bt2.reshape(1, D)
    b3r, g3r, bt3r = b3.reshape(1, D), g3.reshape(1, D), bt3.reshape(1, D)

    degp = _deg_call()(dstw)
    h1, dis = _k1_call(xp, W1, degp.T)
    p1 = _scat_call()(h1, srcw, dstw)
    h2 = _kmid_call(p1, dis, b1r, g1r, bt1r, W2)
    p2 = _scat_call()(h2, srcw, dstw)
    h3 = _kmid_call(p2, dis, b2r, g2r, bt2r, W3)
    p3 = _scat_call()(h3, srcw, dstw)
    return _kfin_call(p3, dis, b3r, g3r, bt3r, batch2d, Wl,
                      bl.reshape(1, NCLS))


# speedup vs baseline: 29.1685x; 1.0353x over previous
import functools

import jax
import jax.numpy as jnp
import numpy as np
from jax import lax
from jax.experimental import pallas as pl
from jax.experimental.pallas import tpu as pltpu
from jax.experimental.pallas import tpu_sc as plsc

N = 10000
NP = 10240
D = 128
E = 320000
G = 64
NCLS = 4
NTILES = 32
WSZ = 128
WPC = 160
WPT = 80
EPT = WPC * WSZ
EPAD = 16 * EPT
DH = D // 2
RPT = NP // 16
NBUF = 6
LEAD = 3

def _deg_body(dst_hbm, out_hbm, dst_v, ones_v, zer_v, acc_s):
    cid = lax.axis_index("c")
    sid = lax.axis_index("s")
    pltpu.sync_copy(dst_hbm.at[sid, pl.ds(cid * WPT, WPT)], dst_v)
    zeros16 = jnp.zeros((16,), jnp.float32)
    for i in range(RPT // 16):
        zer_v[pl.ds(i * 16, 16)] = zeros16
    ones16 = jnp.ones((16,), jnp.float32)
    for i in range(WSZ // 16):
        ones_v[pl.ds(i * 16, 16)] = ones16
    pltpu.sync_copy(zer_v, acc_s.at[pl.ds(sid * RPT, RPT)])
    plsc.subcore_barrier()
    for w in range(WPT):
        pltpu.sync_copy(ones_v, acc_s.at[dst_v.at[np.int32(w)]], add=True)
    plsc.subcore_barrier()
    pltpu.sync_copy(acc_s.at[pl.ds(sid * RPT, RPT)],
                    out_hbm.at[cid, pl.ds(sid * RPT, RPT)])


@functools.cache
def _deg_call():
    mesh = plsc.VectorSubcoreMesh(core_axis_name="c", subcore_axis_name="s",
                                  num_cores=2, num_subcores=16)
    return pl.kernel(
        _deg_body,
        out_type=jax.ShapeDtypeStruct((2, NP), jnp.float32),
        mesh=mesh,
        scratch_types=[
            pltpu.VMEM((WPT, WSZ), jnp.int32),
            pltpu.VMEM((WSZ,), jnp.float32),
            pltpu.VMEM((RPT,), jnp.float32),
            pltpu.VMEM_SHARED((NP,), jnp.float32),
        ],
    )


def _scat_body(hs_hbm, srcw_hbm, dstw_hbm, out_hbm,
               src_v, dst_v, buf_v, gsem_a, ssem_a, acc_s):
    gsems = [gsem_a.at[np.int32(b)] for b in range(NBUF)]
    ssems = [ssem_a.at[np.int32(b)] for b in range(NBUF)]
    cid = lax.axis_index("c")
    sid = lax.axis_index("s")
    hh = hs_hbm.at[cid]
    pltpu.sync_copy(srcw_hbm.at[sid], src_v)
    pltpu.sync_copy(dstw_hbm.at[sid], dst_v)
    pltpu.sync_copy(hh.at[pl.ds(sid * RPT, RPT)],
                    acc_s.at[pl.ds(sid * RPT, RPT)])
    plsc.subcore_barrier()
    gcp = [None] * NBUF
    scp = [None] * NBUF

    def start_gather(w):
        b = w % NBUF
        gcp[b] = pltpu.async_copy(hh.at[src_v.at[np.int32(w)]],
                                  buf_v.at[np.int32(b)], gsems[b])

    for w in range(LEAD):
        start_gather(w)
    for w in range(WPC):
        b = w % NBUF
        nxt = w + LEAD
        if nxt < WPC:
            nb = nxt % NBUF
            if scp[nb] is not None:
                scp[nb].wait()
                scp[nb] = None
            start_gather(nxt)
        gcp[b].wait()
        scp[b] = pltpu.async_copy(buf_v.at[np.int32(b)],
                                  acc_s.at[dst_v.at[np.int32(w)]],
                                  ssems[b], add=True)
    for b in range(NBUF):
        if scp[b] is not None:
            scp[b].wait()
    plsc.subcore_barrier()
    pltpu.sync_copy(acc_s.at[pl.ds(sid * RPT, RPT)],
                    out_hbm.at[pl.ds(sid * RPT, RPT), pl.ds(cid * DH, DH)])


@functools.cache
def _scat_call():
    mesh = plsc.VectorSubcoreMesh(core_axis_name="c", subcore_axis_name="s",
                                  num_cores=2, num_subcores=16)
    return pl.kernel(
        _scat_body,
        out_type=jax.ShapeDtypeStruct((NP, D), jnp.float32),
        mesh=mesh,
        compiler_params=pltpu.CompilerParams(use_tc_tiling_on_sc=False),
        scratch_types=[
            pltpu.VMEM((WPC, WSZ), jnp.int32),
            pltpu.VMEM((WPC, WSZ), jnp.int32),
            pltpu.VMEM((NBUF, WSZ, DH), jnp.float32),
            pltpu.SemaphoreType.DMA((NBUF,)),
            pltpu.SemaphoreType.DMA((NBUF,)),
            pltpu.VMEM_SHARED((NP, DH), jnp.float32),
        ],
    )


def _k1_body(x_ref, w_ref, degp_ref, h_ref, dis_ref):
    d = 1.0 + degp_ref[:, 0:1] + degp_ref[:, 1:2]
    dis = lax.rsqrt(d)
    h = jnp.dot(x_ref[...], w_ref[...],
                preferred_element_type=jnp.float32) * dis
    h_ref[0] = h[:, :DH]
    h_ref[1] = h[:, DH:]
    dis_ref[...] = dis


_k1_call = pl.pallas_call(
    _k1_body,
    out_shape=[jax.ShapeDtypeStruct((2, NP, DH), jnp.float32),
               jax.ShapeDtypeStruct((NP, 1), jnp.float32)],
)


def _bn_relu(p_ref, dis_ref, b_ref, g_ref, bt_ref):
    a = dis_ref[...] * p_ref[...] + b_ref[...]
    mask = (lax.broadcasted_iota(jnp.int32, (NP, 1), 0) < N)
    am = jnp.where(mask, a, 0.0)
    s1 = jnp.sum(am, axis=0, keepdims=True)
    s2 = jnp.sum(am * am, axis=0, keepdims=True)
    mu = s1 * (1.0 / N)
    var = s2 * (1.0 / N) - mu * mu
    return jnp.maximum((a - mu) * lax.rsqrt(var + 1e-5) * g_ref[...]
                       + bt_ref[...], 0.0)


def _kmid_body(p_ref, dis_ref, b_ref, g_ref, bt_ref, w_ref, o_ref):
    r = _bn_relu(p_ref, dis_ref, b_ref, g_ref, bt_ref)
    h = jnp.dot(r, w_ref[...],
                preferred_element_type=jnp.float32) * dis_ref[...]
    o_ref[0] = h[:, :DH]
    o_ref[1] = h[:, DH:]


_kmid_call = pl.pallas_call(
    _kmid_body,
    out_shape=jax.ShapeDtypeStruct((2, NP, DH), jnp.float32),
)


def _kfin_body(p_ref, dis_ref, b_ref, g_ref, bt_ref, batch_ref,
               wl_ref, bl_ref, o_ref):
    r = _bn_relu(p_ref, dis_ref, b_ref, g_ref, bt_ref)
    r10 = r[:N, :]
    oh = (batch_ref[...] ==
          lax.broadcasted_iota(jnp.int32, (G, N), 0)).astype(jnp.float32)
    sums = jnp.dot(oh, r10, preferred_element_type=jnp.float32)
    cnts = jnp.sum(oh, axis=1, keepdims=True)
    pooled = sums / jnp.maximum(cnts, 1.0)
    o_ref[...] = jnp.dot(pooled, wl_ref[...],
                         preferred_element_type=jnp.float32) + bl_ref[...]


_kfin_call = pl.pallas_call(
    _kfin_body,
    out_shape=jax.ShapeDtypeStruct((G, NCLS), jnp.float32),
)


def kernel(x, edge_index, batch, W1, b1, g1, bt1, W2, b2, g2, bt2,
           W3, b3, g3, bt3, Wl, bl):
    src = edge_index[0].astype(jnp.int32)
    dst = edge_index[1].astype(jnp.int32)
    npad = EPAD - E
    ar = jnp.arange(npad, dtype=jnp.int32)
    srcw = jnp.concatenate([src, (ar * 911) % N]).reshape(16, WPC, WSZ)
    dstw = jnp.concatenate([dst, N + ar % (NP - N)]).reshape(16, WPC, WSZ)
    xp = jnp.pad(x, ((0, NP - N), (0, 0)))
    batch2d = batch.astype(jnp.int32).reshape(1, N)
    b1r, g1r, bt1r = b1.reshape(1, D), g1.reshape(1, D), bt1.reshape(1, D)
    b2r, g2r, bt2r = b2.reshape(1, D), g2.reshape(1, D), bt2.reshape(1, D)
    b3r, g3r, bt3r = b3.reshape(1, D), g3.reshape(1, D), bt3.reshape(1, D)

    degp = _deg_call()(dstw)
    h1, dis = _k1_call(xp, W1, degp.T)
    p1 = _scat_call()(h1, srcw, dstw)
    h2 = _kmid_call(p1, dis, b1r, g1r, bt1r, W2)
    p2 = _scat_call()(h2, srcw, dstw)
    h3 = _kmid_call(p2, dis, b2r, g2r, bt2r, W3)
    p3 = _scat_call()(h3, srcw, dstw)
    return _kfin_call(p3, dis, b3r, g3r, bt3r, batch2d, Wl,
                      bl.reshape(1, NCLS))
